# baseline (speedup 1.0000x reference)
import functools

import jax
import jax.numpy as jnp
from jax import lax
from jax.experimental import pallas as pl
from jax.experimental.pallas import tpu as pltpu

_SMOOTHING = 0.1
_CONFIDENCE = 1.0 - _SMOOTHING

_BC = 800


def _loss_kernel(xt_ref, tgt_ref, out_ref, m_acc, s_acc, sum_acc, pt_acc,
                 *, num_classes, num_rows, n_steps):
    i = pl.program_id(0)

    @pl.when(i == 0)
    def _init():
        m_acc[...] = jnp.full((1, num_rows), -jnp.inf, jnp.float32)
        s_acc[...] = jnp.zeros((1, num_rows), jnp.float32)
        sum_acc[...] = jnp.zeros((1, num_rows), jnp.float32)
        pt_acc[...] = jnp.zeros((1, num_rows), jnp.float32)

    t = tgt_ref[...]
    sub_iota = lax.broadcasted_iota(jnp.int32, (8, num_rows), 0)
    neg_inf8 = jnp.full((8, num_rows), -jnp.inf, jnp.float32)
    zeros8 = jnp.zeros((8, num_rows), jnp.float32)

    def p1(k, carry):
        mv, sv, pv = carry
        xk = xt_ref[pl.ds(k * 8, 8), :]
        match = (sub_iota + (i * _BC + k * 8)) == t
        return (jnp.maximum(mv, xk), sv + xk,
                pv + jnp.where(match, xk, 0.0))

    mv, sv, pv = lax.fori_loop(0, _BC // 8, p1, (neg_inf8, zeros8, zeros8),
                               unroll=2)
    m_old = m_acc[...]
    m_new = jnp.maximum(m_old, jnp.max(mv, axis=0, keepdims=True))
    sum_acc[...] += jnp.sum(sv, axis=0, keepdims=True)
    pt_acc[...] += jnp.sum(pv, axis=0, keepdims=True)

    def p2(k, ev):
        xk = xt_ref[pl.ds(k * 8, 8), :]
        return ev + jnp.exp(xk - m_new)

    ev = lax.fori_loop(0, _BC // 8, p2, zeros8, unroll=2)
    s_acc[...] = (s_acc[...] * jnp.exp(m_old - m_new)
                  + jnp.sum(ev, axis=0, keepdims=True))
    m_acc[...] = m_new

    @pl.when(i == n_steps - 1)
    def _finish():
        lse = m_acc[...] + jnp.log(s_acc[...])
        sv = _SMOOTHING / (num_classes - 1)
        loss_rows = -(sv * (sum_acc[...] - num_classes * lse)
                      + (_CONFIDENCE - sv) * (pt_acc[...] - lse))
        out_ref[...] = (jnp.sum(loss_rows) / num_rows).reshape(1, 1)


def kernel(pred, target):
    num_rows, num_classes = pred.shape
    xt = pred.T
    tgt = target.astype(jnp.int32).reshape(1, num_rows)

    n_steps = num_classes // _BC
    out = pl.pallas_call(
        functools.partial(_loss_kernel, num_classes=num_classes,
                          num_rows=num_rows, n_steps=n_steps),
        grid=(n_steps,),
        in_specs=[
            pl.BlockSpec((_BC, num_rows), lambda i: (i, 0)),
            pl.BlockSpec((1, num_rows), lambda i: (0, 0)),
        ],
        out_specs=pl.BlockSpec((1, 1), lambda i: (0, 0)),
        out_shape=jax.ShapeDtypeStruct((1, 1), jnp.float32),
        scratch_shapes=[pltpu.VMEM((1, num_rows), jnp.float32)] * 4,
    )(xt, tgt)
    return out[0, 0]


# speedup vs baseline: 1.7139x; 1.7139x over previous
import functools

import jax
import jax.numpy as jnp
from jax import lax
from jax.experimental import pallas as pl
from jax.experimental.pallas import tpu as pltpu

_SMOOTHING = 0.1
_CONFIDENCE = 1.0 - _SMOOTHING

_BC = 800


def _loss_kernel(xt_ref, tgt_ref, out_ref, m_acc, s_acc, sum_acc, pt_acc,
                 *, num_classes, num_rows, n_steps):
    i = pl.program_id(0)

    @pl.when(i == 0)
    def _init():
        m_acc[...] = jnp.full((1, num_rows), -jnp.inf, jnp.float32)
        s_acc[...] = jnp.zeros((1, num_rows), jnp.float32)
        sum_acc[...] = jnp.zeros((1, num_rows), jnp.float32)
        pt_acc[...] = jnp.zeros((1, num_rows), jnp.float32)

    x = xt_ref[...]
    t = tgt_ref[...]

    m_old = m_acc[...]
    m_new = jnp.maximum(m_old, jnp.max(x, axis=0, keepdims=True))
    e_sum = jnp.sum(jnp.exp(x - m_new), axis=0, keepdims=True)
    s_acc[...] = s_acc[...] * jnp.exp(m_old - m_new) + e_sum
    m_acc[...] = m_new
    sum_acc[...] += jnp.sum(x, axis=0, keepdims=True)

    c_iota = lax.broadcasted_iota(jnp.int32, x.shape, 0) + i * _BC
    pt_acc[...] += jnp.sum(jnp.where(c_iota == t, x, 0.0), axis=0,
                           keepdims=True)

    @pl.when(i == n_steps - 1)
    def _finish():
        lse = m_acc[...] + jnp.log(s_acc[...])
        sv = _SMOOTHING / (num_classes - 1)
        loss_rows = -(sv * (sum_acc[...] - num_classes * lse)
                      + (_CONFIDENCE - sv) * (pt_acc[...] - lse))
        out_ref[...] = (jnp.sum(loss_rows) / num_rows).reshape(1, 1)


def kernel(pred, target):
    num_rows, num_classes = pred.shape
    xt = pred.T
    tgt = target.astype(jnp.int32).reshape(1, num_rows)

    n_steps = num_classes // _BC
    out = pl.pallas_call(
        functools.partial(_loss_kernel, num_classes=num_classes,
                          num_rows=num_rows, n_steps=n_steps),
        grid=(n_steps,),
        in_specs=[
            pl.BlockSpec((_BC, num_rows), lambda i: (i, 0)),
            pl.BlockSpec((1, num_rows), lambda i: (0, 0)),
        ],
        out_specs=pl.BlockSpec((1, 1), lambda i: (0, 0)),
        out_shape=jax.ShapeDtypeStruct((1, 1), jnp.float32),
        scratch_shapes=[pltpu.VMEM((1, num_rows), jnp.float32)] * 4,
    )(xt, tgt)
    return out[0, 0]


# speedup vs baseline: 2.0098x; 1.1726x over previous
import functools

import jax
import jax.numpy as jnp
from jax import lax
from jax.experimental import pallas as pl
from jax.experimental.pallas import tpu as pltpu
from jax.experimental.pallas import tpu_sc as plsc

_SMOOTHING = 0.1
_CONFIDENCE = 1.0 - _SMOOTHING

_BC = 800
_NW = 32
_LANES = 16
_GCHUNK = 16


def _tc_main_kernel(xt_ref, out_ref, m_acc, s_acc, sum_acc,
                    *, num_classes, num_rows, n_steps):
    i = pl.program_id(0)

    @pl.when(i == 0)
    def _init():
        m_acc[...] = jnp.full((1, num_rows), -jnp.inf, jnp.float32)
        s_acc[...] = jnp.zeros((1, num_rows), jnp.float32)
        sum_acc[...] = jnp.zeros((1, num_rows), jnp.float32)

    x = xt_ref[...]

    m_old = m_acc[...]
    m_new = jnp.maximum(m_old, jnp.max(x, axis=0, keepdims=True))
    e_sum = jnp.sum(jnp.exp(x - m_new), axis=0, keepdims=True)
    s_acc[...] = s_acc[...] * jnp.exp(m_old - m_new) + e_sum
    m_acc[...] = m_new
    sum_acc[...] += jnp.sum(x, axis=0, keepdims=True)

    @pl.when(i == n_steps - 1)
    def _finish():
        lse = m_acc[...] + jnp.log(s_acc[...])
        sv = _SMOOTHING / (num_classes - 1)
        a_rows = (-sv * (sum_acc[...] - num_classes * lse)
                  + (_CONFIDENCE - sv) * lse)
        out_ref[...] = jnp.sum(a_rows).reshape(1, 1)


def _sc_gather_body(xt_hbm, target_hbm, pt_hbm, tbuf, rowsbuf, outbuf, sem,
                    *, num_rows, spw):
    wid = lax.axis_index("s") * 2 + lax.axis_index("c")
    base = wid * spw
    pltpu.sync_copy(target_hbm.at[pl.ds(base, spw)], tbuf)
    row_iota = lax.iota(jnp.int32, _LANES)

    def chunk_body(c, carry):
        tvec = tbuf[pl.ds(c * _GCHUNK, _GCHUNK)]
        pltpu.async_copy(xt_hbm.at[tvec], rowsbuf, sem).wait()
        cols = base + c * _GCHUNK + row_iota
        vals = plsc.load_gather(rowsbuf, [row_iota, cols])
        outbuf[pl.ds(c * _GCHUNK, _GCHUNK)] = vals
        return carry

    lax.fori_loop(0, spw // _GCHUNK, chunk_body, 0)
    pltpu.sync_copy(outbuf, pt_hbm.at[pl.ds(base, spw)])


def _combine_kernel(a_ref, pt_ref, out_ref, *, num_classes, num_rows):
    sv = _SMOOTHING / (num_classes - 1)
    pt_total = jnp.sum(pt_ref[...])
    out_ref[...] = ((a_ref[0, 0] - (_CONFIDENCE - sv) * pt_total)
                    / num_rows).reshape(1, 1)


def kernel(pred, target):
    num_rows, num_classes = pred.shape
    xt = pred.T
    tgt = target.astype(jnp.int32)

    n_steps = num_classes // _BC
    a_part = pl.pallas_call(
        functools.partial(_tc_main_kernel, num_classes=num_classes,
                          num_rows=num_rows, n_steps=n_steps),
        grid=(n_steps,),
        in_specs=[pl.BlockSpec((_BC, num_rows), lambda i: (i, 0))],
        out_specs=pl.BlockSpec((1, 1), lambda i: (0, 0)),
        out_shape=jax.ShapeDtypeStruct((1, 1), jnp.float32),
        scratch_shapes=[pltpu.VMEM((1, num_rows), jnp.float32)] * 3,
    )(xt)

    spw = num_rows // _NW
    mesh = plsc.VectorSubcoreMesh(core_axis_name="c", subcore_axis_name="s")
    pt = pl.kernel(
        functools.partial(_sc_gather_body, num_rows=num_rows, spw=spw),
        out_type=jax.ShapeDtypeStruct((num_rows,), jnp.float32),
        mesh=mesh,
        compiler_params=pltpu.CompilerParams(needs_layout_passes=False),
        scratch_types=[
            pltpu.VMEM((spw,), jnp.int32),
            pltpu.VMEM((_GCHUNK, num_rows), jnp.float32),
            pltpu.VMEM((spw,), jnp.float32),
            pltpu.SemaphoreType.DMA,
        ],
    )(xt, tgt)

    rows8 = num_rows // 8
    out = pl.pallas_call(
        functools.partial(_combine_kernel, num_classes=num_classes,
                          num_rows=num_rows),
        in_specs=[
            pl.BlockSpec((1, 1), lambda: (0, 0)),
            pl.BlockSpec((8, rows8), lambda: (0, 0)),
        ],
        out_specs=pl.BlockSpec((1, 1), lambda: (0, 0)),
        out_shape=jax.ShapeDtypeStruct((1, 1), jnp.float32),
    )(a_part, pt.reshape(8, rows8))
    return out[0, 0]


# speedup vs baseline: 2.0604x; 1.0252x over previous
import functools

import jax
import jax.numpy as jnp
from jax import lax
from jax.experimental import pallas as pl
from jax.experimental.pallas import tpu as pltpu
from jax.experimental.pallas import tpu_sc as plsc

_SMOOTHING = 0.1
_CONFIDENCE = 1.0 - _SMOOTHING

_BC = 1000
_NW = 32
_LANES = 16
_GCHUNK = 16


def _tc_main_kernel(xt_ref, out_ref, m_acc, s_acc, sum_acc,
                    *, num_classes, num_rows, n_steps):
    i = pl.program_id(0)

    @pl.when(i == 0)
    def _init():
        m_acc[...] = jnp.full((1, num_rows), -jnp.inf, jnp.float32)
        s_acc[...] = jnp.zeros((1, num_rows), jnp.float32)
        sum_acc[...] = jnp.zeros((1, num_rows), jnp.float32)

    x = xt_ref[...]

    m_old = m_acc[...]
    m_new = jnp.maximum(m_old, jnp.max(x, axis=0, keepdims=True))
    e_sum = jnp.sum(jnp.exp(x - m_new), axis=0, keepdims=True)
    s_acc[...] = s_acc[...] * jnp.exp(m_old - m_new) + e_sum
    m_acc[...] = m_new
    sum_acc[...] += jnp.sum(x, axis=0, keepdims=True)

    @pl.when(i == n_steps - 1)
    def _finish():
        lse = m_acc[...] + jnp.log(s_acc[...])
        sv = _SMOOTHING / (num_classes - 1)
        a_rows = (-sv * (sum_acc[...] - num_classes * lse)
                  + (_CONFIDENCE - sv) * lse)
        out_ref[...] = jnp.sum(a_rows).reshape(1, 1)


def _sc_gather_body(xt_hbm, target_hbm, pt_hbm, tbuf, rowsbuf, outbuf, sem,
                    *, num_rows, spw):
    wid = lax.axis_index("s") * 2 + lax.axis_index("c")
    base = wid * spw
    pltpu.sync_copy(target_hbm.at[pl.ds(base, spw)], tbuf)
    row_iota = lax.iota(jnp.int32, _LANES)

    def chunk_body(c, carry):
        tvec = tbuf[pl.ds(c * _GCHUNK, _GCHUNK)]
        pltpu.async_copy(xt_hbm.at[tvec], rowsbuf, sem).wait()
        cols = base + c * _GCHUNK + row_iota
        vals = plsc.load_gather(rowsbuf, [row_iota, cols])
        outbuf[pl.ds(c * _GCHUNK, _GCHUNK)] = vals
        return carry

    lax.fori_loop(0, spw // _GCHUNK, chunk_body, 0)
    pltpu.sync_copy(outbuf, pt_hbm.at[pl.ds(base, spw)])


def _combine_kernel(a_ref, pt_ref, out_ref, *, num_classes, num_rows):
    sv = _SMOOTHING / (num_classes - 1)
    pt_total = jnp.sum(pt_ref[...])
    out_ref[...] = ((a_ref[0, 0] - (_CONFIDENCE - sv) * pt_total)
                    / num_rows).reshape(1, 1)


def kernel(pred, target):
    num_rows, num_classes = pred.shape
    xt = pred.T
    tgt = target.astype(jnp.int32)

    n_steps = num_classes // _BC
    a_part = pl.pallas_call(
        functools.partial(_tc_main_kernel, num_classes=num_classes,
                          num_rows=num_rows, n_steps=n_steps),
        grid=(n_steps,),
        in_specs=[pl.BlockSpec((_BC, num_rows), lambda i: (i, 0))],
        out_specs=pl.BlockSpec((1, 1), lambda i: (0, 0)),
        out_shape=jax.ShapeDtypeStruct((1, 1), jnp.float32),
        scratch_shapes=[pltpu.VMEM((1, num_rows), jnp.float32)] * 3,
    )(xt)

    spw = num_rows // _NW
    mesh = plsc.VectorSubcoreMesh(core_axis_name="c", subcore_axis_name="s")
    pt = pl.kernel(
        functools.partial(_sc_gather_body, num_rows=num_rows, spw=spw),
        out_type=jax.ShapeDtypeStruct((num_rows,), jnp.float32),
        mesh=mesh,
        compiler_params=pltpu.CompilerParams(needs_layout_passes=False),
        scratch_types=[
            pltpu.VMEM((spw,), jnp.int32),
            pltpu.VMEM((_GCHUNK, num_rows), jnp.float32),
            pltpu.VMEM((spw,), jnp.float32),
            pltpu.SemaphoreType.DMA,
        ],
    )(xt, tgt)

    rows8 = num_rows // 8
    out = pl.pallas_call(
        functools.partial(_combine_kernel, num_classes=num_classes,
                          num_rows=num_rows),
        in_specs=[
            pl.BlockSpec((1, 1), lambda: (0, 0)),
            pl.BlockSpec((8, rows8), lambda: (0, 0)),
        ],
        out_specs=pl.BlockSpec((1, 1), lambda: (0, 0)),
        out_shape=jax.ShapeDtypeStruct((1, 1), jnp.float32),
    )(a_part, pt.reshape(8, rows8))
    return out[0, 0]
